# baseline (speedup 1.0000x reference)
import functools

import jax
import jax.numpy as jnp
from jax import lax
from jax.experimental import pallas as pl
from jax.experimental.pallas import tpu as pltpu
from jax.experimental.pallas import tpu_sc as plsc

E = 8
NG = 4
D = 1024
FF = 512
T = 2048

BTG = 256
NBLK = 24
NPAD = NBLK * BTG
NW = 32
TPW = T // NW
CHD = 32
CHC = 16
BTS = 256


def _sigmoid(x):
    return 1.0 / (1.0 + jnp.exp(-x))


def _rne_bf16_bits(v):
    iv = lax.bitcast_convert_type(v, jnp.int32)
    r = iv + jnp.int32(0x7FFF) + (lax.shift_right_logical(iv, 16) & 1)
    return lax.shift_right_logical(r, 16)


def _router_body(x_ref, rw_ref, pos0_ref, pos1_ref, w0_ref, w1_ref,
                 eid_ref, nact_ref):
    x = x_ref[...]
    rw = rw_ref[...]
    logits = lax.dot_general(x, rw, (((1,), (1,)), ((), ())),
                             preferred_element_type=jnp.float32)
    scores = _sigmoid(logits)

    ie = lax.broadcasted_iota(jnp.int32, (E, NG), 0)
    ig = lax.broadcasted_iota(jnp.int32, (E, NG), 1)
    G = (ie // 2 == ig).astype(jnp.float32)

    gs = lax.dot_general(scores, G, (((1,), (0,)), ((), ())),
                         preferred_element_type=jnp.float32,
                         precision=lax.Precision.HIGHEST)
    i4 = lax.broadcasted_iota(jnp.int32, (T, NG), 1)
    m1 = jnp.max(gs, axis=1, keepdims=True)
    g1 = jnp.min(jnp.where(gs == m1, i4, NG), axis=1, keepdims=True)
    gs2 = jnp.where(i4 == g1, -1.0, gs)
    m2 = jnp.max(gs2, axis=1, keepdims=True)
    g2 = jnp.min(jnp.where(gs2 == m2, i4, NG), axis=1, keepdims=True)
    gmask = jnp.logical_or(i4 == g1, i4 == g2).astype(jnp.float32)

    emask = lax.dot_general(gmask, G, (((1,), (1,)), ((), ())),
                            preferred_element_type=jnp.float32,
                            precision=lax.Precision.HIGHEST)
    sm = jnp.where(emask > 0.5, scores, 0.0)
    i8 = lax.broadcasted_iota(jnp.int32, (T, E), 1)
    s1 = jnp.max(sm, axis=1, keepdims=True)
    e1 = jnp.min(jnp.where(sm == s1, i8, E), axis=1, keepdims=True)
    sm2 = jnp.where(i8 == e1, -1.0, sm)
    s2 = jnp.max(sm2, axis=1, keepdims=True)
    e2 = jnp.min(jnp.where(sm2 == s2, i8, E), axis=1, keepdims=True)

    norm = s1 + s2 + 1e-20
    w1 = s1 / norm
    w2 = s2 / norm
    w0_ref[...] = jnp.broadcast_to(w1, (T, 16))
    w1_ref[...] = jnp.broadcast_to(w2, (T, 16))

    oh2 = ((i8 == e1).astype(jnp.float32)
           + (i8 == e2).astype(jnp.float32))
    incl = oh2
    sh = 1
    while sh < T:
        shifted = jnp.concatenate(
            [jnp.zeros((sh, E), jnp.float32), incl[:T - sh]], axis=0)
        incl = incl + shifted
        sh *= 2
    excl = incl - oh2
    counts = incl[T - 1:T, :]

    cnt_i = counts.astype(jnp.int32)
    cnt_pad = ((cnt_i + BTG - 1) // BTG) * BTG
    ue = lax.broadcasted_iota(jnp.int32, (E, E), 0)
    uf = lax.broadcasted_iota(jnp.int32, (E, E), 1)
    U = (ue < uf).astype(jnp.float32)
    start_pad = lax.dot_general(cnt_pad.astype(jnp.float32), U,
                                (((1,), (0,)), ((), ())),
                                preferred_element_type=jnp.float32,
                                precision=lax.Precision.HIGHEST)

    sel0 = (i8 == e1)
    sel1 = (i8 == e2)
    base_rank = start_pad + excl
    pos0_ref[...] = jnp.sum(jnp.where(sel0, base_rank, 0.0), axis=1,
                            keepdims=True).astype(jnp.int32)
    pos1_ref[...] = jnp.sum(jnp.where(sel1, base_rank, 0.0), axis=1,
                            keepdims=True).astype(jnp.int32)

    start_blk = start_pad.astype(jnp.int32) // BTG
    bb = lax.broadcasted_iota(jnp.int32, (NBLK, E), 0)
    ge = (bb >= start_blk).astype(jnp.int32)
    eid_ref[...] = jnp.sum(ge, axis=1, keepdims=True) - 1
    nact_ref[...] = jnp.sum(cnt_pad, axis=1, keepdims=True) // BTG


def _router_call(x, router_w):
    return pl.pallas_call(
        _router_body,
        out_shape=[
            jax.ShapeDtypeStruct((T, 1), jnp.int32),
            jax.ShapeDtypeStruct((T, 1), jnp.int32),
            jax.ShapeDtypeStruct((T, 16), jnp.float32),
            jax.ShapeDtypeStruct((T, 16), jnp.float32),
            jax.ShapeDtypeStruct((NBLK, 1), jnp.int32),
            jax.ShapeDtypeStruct((1, 1), jnp.int32),
        ],
        in_specs=[
            pl.BlockSpec((T, D), lambda: (0, 0)),
            pl.BlockSpec((E, D), lambda: (0, 0)),
        ],
        out_specs=[
            pl.BlockSpec((T, 1), lambda: (0, 0)),
            pl.BlockSpec((T, 1), lambda: (0, 0)),
            pl.BlockSpec((T, 16), lambda: (0, 0)),
            pl.BlockSpec((T, 16), lambda: (0, 0)),
            pl.BlockSpec((NBLK, 1), lambda: (0, 0)),
            pl.BlockSpec((1, 1), lambda: (0, 0)),
        ],
    )(x, router_w)


def _dispatch_body(x_hbm, pos0_hbm, pos1_hbm, xs_hbm, xbuf, idx0, idx1, sem):
    wid = lax.axis_index("s") * 2 + lax.axis_index("c")
    pltpu.sync_copy(pos0_hbm.at[wid], idx0)
    pltpu.sync_copy(pos1_hbm.at[wid], idx1)
    pltpu.sync_copy(x_hbm.at[pl.ds(wid * TPW, TPW)], xbuf)
    cp0 = pltpu.async_copy(xbuf, xs_hbm.at[idx0], sem)
    cp1 = pltpu.async_copy(xbuf, xs_hbm.at[idx1], sem)
    cp0.wait()
    cp1.wait()


def _dispatch_call():
    return pl.kernel(
        _dispatch_body,
        out_type=jax.ShapeDtypeStruct((NPAD, D), jnp.float32),
        mesh=plsc.VectorSubcoreMesh(core_axis_name="c", subcore_axis_name="s"),
        scratch_types=[
            pltpu.VMEM((TPW, D), jnp.float32),
            pltpu.VMEM((TPW,), jnp.int32),
            pltpu.VMEM((TPW,), jnp.int32),
            pltpu.SemaphoreType.DMA,
        ],
        compiler_params=pltpu.CompilerParams(disable_bounds_checks=True),
    )


def _grouped_body(eid_ref, nact_ref, xs_ref, g_ref, u_ref, d_ref, ys_ref,
                  gbf, ubf, dbf, acc):
    i = pl.program_id(0)
    k = pl.program_id(1)

    @pl.when(i < nact_ref[0])
    def _():
        prev = eid_ref[jnp.maximum(i - 1, 0)]
        is_new = jnp.logical_or(i == 0, eid_ref[i] != prev)

        @pl.when(is_new)
        def _():
            ksl = pl.ds(k * (FF // 2), FF // 2)
            gbf[ksl, :] = g_ref[0].astype(jnp.bfloat16)
            ubf[ksl, :] = u_ref[0].astype(jnp.bfloat16)
            dbf[:, ksl] = d_ref[0].astype(jnp.bfloat16)

        ksl = pl.ds(k * (FF // 2), FF // 2)
        xs = xs_ref[...].astype(jnp.bfloat16)
        gate = lax.dot_general(xs, gbf[ksl, :], (((1,), (1,)), ((), ())),
                               preferred_element_type=jnp.float32)
        up = lax.dot_general(xs, ubf[ksl, :], (((1,), (1,)), ((), ())),
                             preferred_element_type=jnp.float32)
        h = (gate * _sigmoid(gate) * up).astype(jnp.bfloat16)
        y = lax.dot_general(h, dbf[:, ksl], (((1,), (1,)), ((), ())),
                            preferred_element_type=jnp.float32)

        @pl.when(k == 0)
        def _():
            acc[...] = y

        @pl.when(k == 1)
        def _():
            yt = acc[...] + y
            ilo = _rne_bf16_bits(yt[:, :D // 2])
            ihi = _rne_bf16_bits(yt[:, D // 2:])
            ys_ref[...] = ilo | (ihi << 16)


def _grouped_call(eid, nact, xs, gate_w, up_w, down_w):
    grid_spec = pltpu.PrefetchScalarGridSpec(
        num_scalar_prefetch=2,
        grid=(NBLK, 2),
        in_specs=[
            pl.BlockSpec((BTG, D),
                         lambda i, k, eid, nact:
                         (jnp.minimum(i, nact[0] - 1), 0)),
            pl.BlockSpec((1, FF // 2, D),
                         lambda i, k, eid, nact: (eid[i], k, 0)),
            pl.BlockSpec((1, FF // 2, D),
                         lambda i, k, eid, nact: (eid[i], k, 0)),
            pl.BlockSpec((1, D, FF // 2),
                         lambda i, k, eid, nact: (eid[i], 0, k)),
        ],
        out_specs=pl.BlockSpec((BTG, D // 2),
                               lambda i, k, eid, nact: (i, 0)),
        scratch_shapes=[
            pltpu.VMEM((FF, D), jnp.bfloat16),
            pltpu.VMEM((FF, D), jnp.bfloat16),
            pltpu.VMEM((D, FF), jnp.bfloat16),
            pltpu.VMEM((BTG, D), jnp.float32),
        ],
    )
    return pl.pallas_call(
        _grouped_body,
        grid_spec=grid_spec,
        out_shape=jax.ShapeDtypeStruct((NPAD, D // 2), jnp.int32),
        compiler_params=pltpu.CompilerParams(
            dimension_semantics=("arbitrary", "arbitrary")),
    )(eid, nact, xs, gate_w, up_w, down_w)


def _shared_body(x_ref, g_ref, u_ref, d_ref, o_ref, gbf, ubf, dbf):
    t = pl.program_id(0)

    @pl.when(t == 0)
    def _():
        gbf[...] = g_ref[...].astype(jnp.bfloat16)
        ubf[...] = u_ref[...].astype(jnp.bfloat16)
        dbf[...] = d_ref[...].astype(jnp.bfloat16)

    x = x_ref[...].astype(jnp.bfloat16)
    gate = lax.dot_general(x, gbf[...], (((1,), (1,)), ((), ())),
                           preferred_element_type=jnp.float32)
    up = lax.dot_general(x, ubf[...], (((1,), (1,)), ((), ())),
                         preferred_element_type=jnp.float32)
    h = (gate * _sigmoid(gate) * up).astype(jnp.bfloat16)
    o_ref[...] = lax.dot_general(h, dbf[...], (((1,), (1,)), ((), ())),
                                 preferred_element_type=jnp.float32)


def _shared_call(x, sh_gate, sh_up, sh_down):
    FFS = sh_gate.shape[0]
    return pl.pallas_call(
        _shared_body,
        grid=(T // BTS,),
        out_shape=jax.ShapeDtypeStruct((T, D), jnp.float32),
        in_specs=[
            pl.BlockSpec((BTS, D), lambda t: (t, 0)),
            pl.BlockSpec((FFS, D), lambda t: (0, 0)),
            pl.BlockSpec((FFS, D), lambda t: (0, 0)),
            pl.BlockSpec((D, FFS), lambda t: (0, 0)),
        ],
        out_specs=pl.BlockSpec((BTS, D), lambda t: (t, 0)),
        scratch_shapes=[
            pltpu.VMEM((FFS, D), jnp.bfloat16),
            pltpu.VMEM((FFS, D), jnp.bfloat16),
            pltpu.VMEM((D, FFS), jnp.bfloat16),
        ],
        compiler_params=pltpu.CompilerParams(
            dimension_semantics=("arbitrary",)),
    )(x, sh_gate, sh_up, sh_down)


def _combine_body(ys_hbm, sh_hbm, pos0_hbm, pos1_hbm, w0_hbm, w1_hbm,
                  out_hbm, y0buf, y1buf, shbuf, w0buf, w1buf, idx0, idx1,
                  semA, semB):
    wid = lax.axis_index("s") * 2 + lax.axis_index("c")
    nchunks = TPW // CHC
    sems = (semA, semB)

    def issue(c, p):
        sem = sems[p]
        pltpu.sync_copy(pos0_hbm.at[wid, pl.ds(c * CHC, CHC)], idx0.at[p])
        pltpu.sync_copy(pos1_hbm.at[wid, pl.ds(c * CHC, CHC)], idx1.at[p])
        base = wid * TPW + c * CHC
        return [
            pltpu.async_copy(ys_hbm.at[idx0.at[p]], y0buf.at[p], sem),
            pltpu.async_copy(ys_hbm.at[idx1.at[p]], y1buf.at[p], sem),
            pltpu.async_copy(sh_hbm.at[pl.ds(base, CHC)], shbuf.at[p], sem),
            pltpu.async_copy(w0_hbm.at[pl.ds(base, CHC)], w0buf.at[p], sem),
            pltpu.async_copy(w1_hbm.at[pl.ds(base, CHC)], w1buf.at[p], sem),
        ]

    cps = issue(0, 0)
    for c in range(nchunks):
        p = c % 2
        for cp in cps:
            cp.wait()
        nxt = issue(c + 1, 1 - p) if c + 1 < nchunks else None

        @plsc.parallel_loop(0, CHC, 1, unroll=4)
        def _(j, p=p):
            w0v = w0buf[p, j]
            w1v = w1buf[p, j]
            for ci in range(D // 32):
                sl = pl.ds(ci * 16, 16)
                slo = pl.ds(ci * 16, 16)
                shi = pl.ds(D // 2 + ci * 16, 16)
                v0 = y0buf[p, j, sl]
                v1 = y1buf[p, j, sl]
                y0lo = plsc.bitcast(lax.shift_left(v0, 16), jnp.float32)
                y1lo = plsc.bitcast(lax.shift_left(v1, 16), jnp.float32)
                y0hi = plsc.bitcast(v0 & jnp.int32(-65536), jnp.float32)
                y1hi = plsc.bitcast(v1 & jnp.int32(-65536), jnp.float32)
                shbuf[p, j, slo] = (y0lo * w0v + y1lo * w1v
                                    + shbuf[p, j, slo])
                shbuf[p, j, shi] = (y0hi * w0v + y1hi * w1v
                                    + shbuf[p, j, shi])
        base = wid * TPW + c * CHC
        pltpu.sync_copy(shbuf.at[p], out_hbm.at[pl.ds(base, CHC)])
        cps = nxt


def _combine_call():
    return pl.kernel(
        _combine_body,
        out_type=jax.ShapeDtypeStruct((T, D), jnp.float32),
        mesh=plsc.VectorSubcoreMesh(core_axis_name="c", subcore_axis_name="s"),
        scratch_types=[
            pltpu.VMEM((2, CHC, D // 2), jnp.int32),
            pltpu.VMEM((2, CHC, D // 2), jnp.int32),
            pltpu.VMEM((2, CHC, D), jnp.float32),
            pltpu.VMEM((2, CHC, 16), jnp.float32),
            pltpu.VMEM((2, CHC, 16), jnp.float32),
            pltpu.VMEM((2, CHC), jnp.int32),
            pltpu.VMEM((2, CHC), jnp.int32),
            pltpu.SemaphoreType.DMA,
            pltpu.SemaphoreType.DMA,
        ],
        compiler_params=pltpu.CompilerParams(needs_layout_passes=False,
                                             disable_bounds_checks=True),
    )


def kernel(hidden_states, router_w, gate_w, up_w, down_w, sh_gate, sh_up,
           sh_down):
    x = hidden_states.reshape(T, D)

    pos0, pos1, w0b, w1b, eid, nact = _router_call(x, router_w)
    pos0r = pos0.reshape(NW, TPW)
    pos1r = pos1.reshape(NW, TPW)

    xs = _dispatch_call()(x, pos0r, pos1r)
    ys = _grouped_call(eid.reshape(NBLK), nact.reshape(1), xs,
                       gate_w, up_w, down_w)
    sh = _shared_call(x, sh_gate, sh_up, sh_down)
    out = _combine_call()(ys, sh, pos0r, pos1r, w0b, w1b)

    return out.reshape(hidden_states.shape)


# speedup vs baseline: 1.2146x; 1.2146x over previous
import functools

import jax
import jax.numpy as jnp
from jax import lax
from jax.experimental import pallas as pl
from jax.experimental.pallas import tpu as pltpu
from jax.experimental.pallas import tpu_sc as plsc

E = 8
NG = 4
D = 1024
FF = 512
T = 2048

BTG = 256
NBLK = 24
NPAD = NBLK * BTG
NW = 32
TPW = T // NW
CHD = 32
CHC = 16
BTS = 256


def _sigmoid(x):
    return 1.0 / (1.0 + jnp.exp(-x))


def _rne_bf16_bits(v):
    iv = lax.bitcast_convert_type(v, jnp.int32)
    r = iv + jnp.int32(0x7FFF) + (lax.shift_right_logical(iv, 16) & 1)
    return lax.shift_right_logical(r, 16)


def _router_body(x_ref, rw_ref, pos0_ref, pos1_ref, w0_ref, w1_ref,
                 eid_ref, nact_ref):
    x = x_ref[...]
    rw = rw_ref[...]
    logits = lax.dot_general(x, rw, (((1,), (1,)), ((), ())),
                             preferred_element_type=jnp.float32)
    scores = _sigmoid(logits)

    ie = lax.broadcasted_iota(jnp.int32, (E, NG), 0)
    ig = lax.broadcasted_iota(jnp.int32, (E, NG), 1)
    G = (ie // 2 == ig).astype(jnp.float32)

    gs = lax.dot_general(scores, G, (((1,), (0,)), ((), ())),
                         preferred_element_type=jnp.float32,
                         precision=lax.Precision.HIGHEST)
    i4 = lax.broadcasted_iota(jnp.int32, (T, NG), 1)
    m1 = jnp.max(gs, axis=1, keepdims=True)
    g1 = jnp.min(jnp.where(gs == m1, i4, NG), axis=1, keepdims=True)
    gs2 = jnp.where(i4 == g1, -1.0, gs)
    m2 = jnp.max(gs2, axis=1, keepdims=True)
    g2 = jnp.min(jnp.where(gs2 == m2, i4, NG), axis=1, keepdims=True)
    gmask = jnp.logical_or(i4 == g1, i4 == g2).astype(jnp.float32)

    emask = lax.dot_general(gmask, G, (((1,), (1,)), ((), ())),
                            preferred_element_type=jnp.float32,
                            precision=lax.Precision.HIGHEST)
    sm = jnp.where(emask > 0.5, scores, 0.0)
    i8 = lax.broadcasted_iota(jnp.int32, (T, E), 1)
    s1 = jnp.max(sm, axis=1, keepdims=True)
    e1 = jnp.min(jnp.where(sm == s1, i8, E), axis=1, keepdims=True)
    sm2 = jnp.where(i8 == e1, -1.0, sm)
    s2 = jnp.max(sm2, axis=1, keepdims=True)
    e2 = jnp.min(jnp.where(sm2 == s2, i8, E), axis=1, keepdims=True)

    norm = s1 + s2 + 1e-20
    w1 = s1 / norm
    w2 = s2 / norm
    w0_ref[...] = jnp.broadcast_to(w1, (T, 16))
    w1_ref[...] = jnp.broadcast_to(w2, (T, 16))

    oh2 = ((i8 == e1).astype(jnp.float32)
           + (i8 == e2).astype(jnp.float32))
    incl = oh2
    sh = 1
    while sh < T:
        shifted = jnp.concatenate(
            [jnp.zeros((sh, E), jnp.float32), incl[:T - sh]], axis=0)
        incl = incl + shifted
        sh *= 2
    excl = incl - oh2
    counts = incl[T - 1:T, :]

    cnt_i = counts.astype(jnp.int32)
    cnt_pad = ((cnt_i + BTG - 1) // BTG) * BTG
    ue = lax.broadcasted_iota(jnp.int32, (E, E), 0)
    uf = lax.broadcasted_iota(jnp.int32, (E, E), 1)
    U = (ue < uf).astype(jnp.float32)
    start_pad = lax.dot_general(cnt_pad.astype(jnp.float32), U,
                                (((1,), (0,)), ((), ())),
                                preferred_element_type=jnp.float32,
                                precision=lax.Precision.HIGHEST)

    sel0 = (i8 == e1)
    sel1 = (i8 == e2)
    base_rank = start_pad + excl
    pos0_ref[...] = jnp.sum(jnp.where(sel0, base_rank, 0.0), axis=1,
                            keepdims=True).astype(jnp.int32)
    pos1_ref[...] = jnp.sum(jnp.where(sel1, base_rank, 0.0), axis=1,
                            keepdims=True).astype(jnp.int32)

    start_blk = start_pad.astype(jnp.int32) // BTG
    bb = lax.broadcasted_iota(jnp.int32, (NBLK, E), 0)
    ge = (bb >= start_blk).astype(jnp.int32)
    eid_ref[...] = jnp.sum(ge, axis=1, keepdims=True) - 1
    nact_ref[...] = jnp.sum(cnt_pad, axis=1, keepdims=True) // BTG


def _router_call(x, router_w):
    return pl.pallas_call(
        _router_body,
        out_shape=[
            jax.ShapeDtypeStruct((T, 1), jnp.int32),
            jax.ShapeDtypeStruct((T, 1), jnp.int32),
            jax.ShapeDtypeStruct((T, 16), jnp.float32),
            jax.ShapeDtypeStruct((T, 16), jnp.float32),
            jax.ShapeDtypeStruct((NBLK, 1), jnp.int32),
            jax.ShapeDtypeStruct((1, 1), jnp.int32),
        ],
        in_specs=[
            pl.BlockSpec((T, D), lambda: (0, 0)),
            pl.BlockSpec((E, D), lambda: (0, 0)),
        ],
        out_specs=[
            pl.BlockSpec((T, 1), lambda: (0, 0)),
            pl.BlockSpec((T, 1), lambda: (0, 0)),
            pl.BlockSpec((T, 16), lambda: (0, 0)),
            pl.BlockSpec((T, 16), lambda: (0, 0)),
            pl.BlockSpec((NBLK, 1), lambda: (0, 0)),
            pl.BlockSpec((1, 1), lambda: (0, 0)),
        ],
    )(x, router_w)


def _dispatch_body(x_hbm, pos0_hbm, pos1_hbm, xs_hbm, xbuf, idx0, idx1, sem):
    wid = lax.axis_index("s") * 2 + lax.axis_index("c")
    pltpu.sync_copy(pos0_hbm.at[wid], idx0)
    pltpu.sync_copy(pos1_hbm.at[wid], idx1)
    pltpu.sync_copy(x_hbm.at[pl.ds(wid * TPW, TPW)], xbuf)
    cp0 = pltpu.async_copy(xbuf, xs_hbm.at[idx0], sem)
    cp1 = pltpu.async_copy(xbuf, xs_hbm.at[idx1], sem)
    cp0.wait()
    cp1.wait()


def _dispatch_call():
    return pl.kernel(
        _dispatch_body,
        out_type=jax.ShapeDtypeStruct((NPAD, D), jnp.float32),
        mesh=plsc.VectorSubcoreMesh(core_axis_name="c", subcore_axis_name="s"),
        scratch_types=[
            pltpu.VMEM((TPW, D), jnp.float32),
            pltpu.VMEM((TPW,), jnp.int32),
            pltpu.VMEM((TPW,), jnp.int32),
            pltpu.SemaphoreType.DMA,
        ],
        compiler_params=pltpu.CompilerParams(disable_bounds_checks=True),
    )


def _grouped_body(eid_ref, nact_ref, xs_ref, g_ref, u_ref, d_ref, ys_ref,
                  gbf, ubf, dbf):
    i = pl.program_id(0)

    @pl.when(i < nact_ref[0])
    def _():
        prev = eid_ref[jnp.maximum(i - 1, 0)]
        is_new = jnp.logical_or(i == 0, eid_ref[i] != prev)

        @pl.when(is_new)
        def _():
            gbf[...] = g_ref[0].astype(jnp.bfloat16)
            ubf[...] = u_ref[0].astype(jnp.bfloat16)
            dbf[...] = d_ref[0].astype(jnp.bfloat16)

        xs = xs_ref[...].astype(jnp.bfloat16)
        gate = lax.dot_general(xs, gbf[...], (((1,), (1,)), ((), ())),
                               preferred_element_type=jnp.float32)
        up = lax.dot_general(xs, ubf[...], (((1,), (1,)), ((), ())),
                             preferred_element_type=jnp.float32)
        h = (gate * _sigmoid(gate) * up).astype(jnp.bfloat16)
        y = lax.dot_general(h, dbf[...], (((1,), (1,)), ((), ())),
                            preferred_element_type=jnp.float32)
        ilo = _rne_bf16_bits(y[:, :D // 2])
        ihi = _rne_bf16_bits(y[:, D // 2:])
        ys_ref[...] = ilo | (ihi << 16)


def _grouped_call(eid, nact, xs, gate_w, up_w, down_w):
    grid_spec = pltpu.PrefetchScalarGridSpec(
        num_scalar_prefetch=2,
        grid=(NBLK,),
        in_specs=[
            pl.BlockSpec((BTG, D),
                         lambda i, eid, nact: (jnp.minimum(i, nact[0] - 1), 0)),
            pl.BlockSpec((1, FF, D), lambda i, eid, nact: (eid[i], 0, 0)),
            pl.BlockSpec((1, FF, D), lambda i, eid, nact: (eid[i], 0, 0)),
            pl.BlockSpec((1, D, FF), lambda i, eid, nact: (eid[i], 0, 0)),
        ],
        out_specs=pl.BlockSpec((BTG, D // 2), lambda i, eid, nact: (i, 0)),
        scratch_shapes=[
            pltpu.VMEM((FF, D), jnp.bfloat16),
            pltpu.VMEM((FF, D), jnp.bfloat16),
            pltpu.VMEM((D, FF), jnp.bfloat16),
        ],
    )
    return pl.pallas_call(
        _grouped_body,
        grid_spec=grid_spec,
        out_shape=jax.ShapeDtypeStruct((NPAD, D // 2), jnp.int32),
        compiler_params=pltpu.CompilerParams(
            dimension_semantics=("arbitrary",)),
    )(eid, nact, xs, gate_w, up_w, down_w)


def _shared_body(x_ref, g_ref, u_ref, d_ref, o_ref, gbf, ubf, dbf):
    t = pl.program_id(0)

    @pl.when(t == 0)
    def _():
        gbf[...] = g_ref[...].astype(jnp.bfloat16)
        ubf[...] = u_ref[...].astype(jnp.bfloat16)
        dbf[...] = d_ref[...].astype(jnp.bfloat16)

    x = x_ref[...].astype(jnp.bfloat16)
    gate = lax.dot_general(x, gbf[...], (((1,), (1,)), ((), ())),
                           preferred_element_type=jnp.float32)
    up = lax.dot_general(x, ubf[...], (((1,), (1,)), ((), ())),
                         preferred_element_type=jnp.float32)
    h = (gate * _sigmoid(gate) * up).astype(jnp.bfloat16)
    o_ref[...] = lax.dot_general(h, dbf[...], (((1,), (1,)), ((), ())),
                                 preferred_element_type=jnp.float32)


def _shared_call(x, sh_gate, sh_up, sh_down):
    FFS = sh_gate.shape[0]
    return pl.pallas_call(
        _shared_body,
        grid=(T // BTS,),
        out_shape=jax.ShapeDtypeStruct((T, D), jnp.float32),
        in_specs=[
            pl.BlockSpec((BTS, D), lambda t: (t, 0)),
            pl.BlockSpec((FFS, D), lambda t: (0, 0)),
            pl.BlockSpec((FFS, D), lambda t: (0, 0)),
            pl.BlockSpec((D, FFS), lambda t: (0, 0)),
        ],
        out_specs=pl.BlockSpec((BTS, D), lambda t: (t, 0)),
        scratch_shapes=[
            pltpu.VMEM((FFS, D), jnp.bfloat16),
            pltpu.VMEM((FFS, D), jnp.bfloat16),
            pltpu.VMEM((D, FFS), jnp.bfloat16),
        ],
        compiler_params=pltpu.CompilerParams(
            dimension_semantics=("arbitrary",)),
    )(x, sh_gate, sh_up, sh_down)


def _combine_body(ys_hbm, sh_hbm, pos0_hbm, pos1_hbm, w0_hbm, w1_hbm,
                  out_hbm, y0buf, y1buf, shbuf, w0buf, w1buf, idx0, idx1,
                  semA, semB):
    wid = lax.axis_index("s") * 2 + lax.axis_index("c")
    nchunks = TPW // CHC
    sems = (semA, semB)

    def issue(c, p):
        sem = sems[p]
        pltpu.sync_copy(pos0_hbm.at[wid, pl.ds(c * CHC, CHC)], idx0.at[p])
        pltpu.sync_copy(pos1_hbm.at[wid, pl.ds(c * CHC, CHC)], idx1.at[p])
        base = wid * TPW + c * CHC
        return [
            pltpu.async_copy(ys_hbm.at[idx0.at[p]], y0buf.at[p], sem),
            pltpu.async_copy(ys_hbm.at[idx1.at[p]], y1buf.at[p], sem),
            pltpu.async_copy(sh_hbm.at[pl.ds(base, CHC)], shbuf.at[p], sem),
            pltpu.async_copy(w0_hbm.at[pl.ds(base, CHC)], w0buf.at[p], sem),
            pltpu.async_copy(w1_hbm.at[pl.ds(base, CHC)], w1buf.at[p], sem),
        ]

    cps = issue(0, 0)
    for c in range(nchunks):
        p = c % 2
        for cp in cps:
            cp.wait()
        nxt = issue(c + 1, 1 - p) if c + 1 < nchunks else None

        @plsc.parallel_loop(0, CHC, 1, unroll=4)
        def _(j, p=p):
            w0v = w0buf[p, j]
            w1v = w1buf[p, j]
            for ci in range(D // 32):
                sl = pl.ds(ci * 16, 16)
                slo = pl.ds(ci * 16, 16)
                shi = pl.ds(D // 2 + ci * 16, 16)
                v0 = y0buf[p, j, sl]
                v1 = y1buf[p, j, sl]
                y0lo = plsc.bitcast(lax.shift_left(v0, 16), jnp.float32)
                y1lo = plsc.bitcast(lax.shift_left(v1, 16), jnp.float32)
                y0hi = plsc.bitcast(v0 & jnp.int32(-65536), jnp.float32)
                y1hi = plsc.bitcast(v1 & jnp.int32(-65536), jnp.float32)
                shbuf[p, j, slo] = (y0lo * w0v + y1lo * w1v
                                    + shbuf[p, j, slo])
                shbuf[p, j, shi] = (y0hi * w0v + y1hi * w1v
                                    + shbuf[p, j, shi])
        base = wid * TPW + c * CHC
        pltpu.sync_copy(shbuf.at[p], out_hbm.at[pl.ds(base, CHC)])
        cps = nxt


def _combine_call():
    return pl.kernel(
        _combine_body,
        out_type=jax.ShapeDtypeStruct((T, D), jnp.float32),
        mesh=plsc.VectorSubcoreMesh(core_axis_name="c", subcore_axis_name="s"),
        scratch_types=[
            pltpu.VMEM((2, CHC, D // 2), jnp.int32),
            pltpu.VMEM((2, CHC, D // 2), jnp.int32),
            pltpu.VMEM((2, CHC, D), jnp.float32),
            pltpu.VMEM((2, CHC, 16), jnp.float32),
            pltpu.VMEM((2, CHC, 16), jnp.float32),
            pltpu.VMEM((2, CHC), jnp.int32),
            pltpu.VMEM((2, CHC), jnp.int32),
            pltpu.SemaphoreType.DMA,
            pltpu.SemaphoreType.DMA,
        ],
        compiler_params=pltpu.CompilerParams(needs_layout_passes=False,
                                             disable_bounds_checks=True),
    )


def kernel(hidden_states, router_w, gate_w, up_w, down_w, sh_gate, sh_up,
           sh_down):
    x = hidden_states.reshape(T, D)

    pos0, pos1, w0b, w1b, eid, nact = _router_call(x, router_w)
    pos0r = pos0.reshape(NW, TPW)
    pos1r = pos1.reshape(NW, TPW)

    xs = _dispatch_call()(x, pos0r, pos1r)
    ys = _grouped_call(eid.reshape(NBLK), nact.reshape(1), xs,
                       gate_w, up_w, down_w)
    sh = _shared_call(x, sh_gate, sh_up, sh_down)
    out = _combine_call()(ys, sh, pos0r, pos1r, w0b, w1b)

    return out.reshape(hidden_states.shape)


# speedup vs baseline: 1.2247x; 1.0083x over previous
import functools

import jax
import jax.numpy as jnp
from jax import lax
from jax.experimental import pallas as pl
from jax.experimental.pallas import tpu as pltpu
from jax.experimental.pallas import tpu_sc as plsc

E = 8
NG = 4
D = 1024
FF = 512
T = 2048

BTG = 256
NBLK = 24
NPAD = NBLK * BTG
NW = 32
TPW = T // NW
CHD = 32
CHC = 16
BTS = 256


def _sigmoid(x):
    return 1.0 / (1.0 + jnp.exp(-x))


def _rne_bf16_bits(v):
    iv = lax.bitcast_convert_type(v, jnp.int32)
    r = iv + jnp.int32(0x7FFF) + (lax.shift_right_logical(iv, 16) & 1)
    return lax.shift_right_logical(r, 16)


def _router_body(x_ref, rw_ref, pos0_ref, pos1_ref, w0_ref, w1_ref,
                 eid_ref, nact_ref):
    x = x_ref[...]
    rw = rw_ref[...]
    logits = lax.dot_general(x, rw, (((1,), (1,)), ((), ())),
                             preferred_element_type=jnp.float32)
    scores = _sigmoid(logits)

    ie = lax.broadcasted_iota(jnp.int32, (E, NG), 0)
    ig = lax.broadcasted_iota(jnp.int32, (E, NG), 1)
    G = (ie // 2 == ig).astype(jnp.float32)

    gs = lax.dot_general(scores, G, (((1,), (0,)), ((), ())),
                         preferred_element_type=jnp.float32,
                         precision=lax.Precision.HIGHEST)
    i4 = lax.broadcasted_iota(jnp.int32, (T, NG), 1)
    m1 = jnp.max(gs, axis=1, keepdims=True)
    g1 = jnp.min(jnp.where(gs == m1, i4, NG), axis=1, keepdims=True)
    gs2 = jnp.where(i4 == g1, -1.0, gs)
    m2 = jnp.max(gs2, axis=1, keepdims=True)
    g2 = jnp.min(jnp.where(gs2 == m2, i4, NG), axis=1, keepdims=True)
    gmask = jnp.logical_or(i4 == g1, i4 == g2).astype(jnp.float32)

    emask = lax.dot_general(gmask, G, (((1,), (1,)), ((), ())),
                            preferred_element_type=jnp.float32,
                            precision=lax.Precision.HIGHEST)
    sm = jnp.where(emask > 0.5, scores, 0.0)
    i8 = lax.broadcasted_iota(jnp.int32, (T, E), 1)
    s1 = jnp.max(sm, axis=1, keepdims=True)
    e1 = jnp.min(jnp.where(sm == s1, i8, E), axis=1, keepdims=True)
    sm2 = jnp.where(i8 == e1, -1.0, sm)
    s2 = jnp.max(sm2, axis=1, keepdims=True)
    e2 = jnp.min(jnp.where(sm2 == s2, i8, E), axis=1, keepdims=True)

    norm = s1 + s2 + 1e-20
    w1 = s1 / norm
    w2 = s2 / norm
    w0_ref[...] = jnp.broadcast_to(w1, (T, 16))
    w1_ref[...] = jnp.broadcast_to(w2, (T, 16))

    oh2 = ((i8 == e1).astype(jnp.float32)
           + (i8 == e2).astype(jnp.float32))
    incl = oh2
    sh = 1
    while sh < T:
        shifted = jnp.concatenate(
            [jnp.zeros((sh, E), jnp.float32), incl[:T - sh]], axis=0)
        incl = incl + shifted
        sh *= 2
    excl = incl - oh2
    counts = incl[T - 1:T, :]

    cnt_i = counts.astype(jnp.int32)
    cnt_pad = ((cnt_i + BTG - 1) // BTG) * BTG
    ue = lax.broadcasted_iota(jnp.int32, (E, E), 0)
    uf = lax.broadcasted_iota(jnp.int32, (E, E), 1)
    U = (ue < uf).astype(jnp.float32)
    start_pad = lax.dot_general(cnt_pad.astype(jnp.float32), U,
                                (((1,), (0,)), ((), ())),
                                preferred_element_type=jnp.float32,
                                precision=lax.Precision.HIGHEST)

    sel0 = (i8 == e1)
    sel1 = (i8 == e2)
    base_rank = start_pad + excl
    pos0_ref[...] = jnp.sum(jnp.where(sel0, base_rank, 0.0), axis=1,
                            keepdims=True).astype(jnp.int32)
    pos1_ref[...] = jnp.sum(jnp.where(sel1, base_rank, 0.0), axis=1,
                            keepdims=True).astype(jnp.int32)

    start_blk = start_pad.astype(jnp.int32) // BTG
    bb = lax.broadcasted_iota(jnp.int32, (NBLK, E), 0)
    ge = (bb >= start_blk).astype(jnp.int32)
    eid_ref[...] = jnp.sum(ge, axis=1, keepdims=True) - 1
    nact_ref[...] = jnp.sum(cnt_pad, axis=1, keepdims=True) // BTG


def _router_call(x, router_w):
    return pl.pallas_call(
        _router_body,
        out_shape=[
            jax.ShapeDtypeStruct((T, 1), jnp.int32),
            jax.ShapeDtypeStruct((T, 1), jnp.int32),
            jax.ShapeDtypeStruct((T, 16), jnp.float32),
            jax.ShapeDtypeStruct((T, 16), jnp.float32),
            jax.ShapeDtypeStruct((NBLK, 1), jnp.int32),
            jax.ShapeDtypeStruct((1, 1), jnp.int32),
        ],
        in_specs=[
            pl.BlockSpec((T, D), lambda: (0, 0)),
            pl.BlockSpec((E, D), lambda: (0, 0)),
        ],
        out_specs=[
            pl.BlockSpec((T, 1), lambda: (0, 0)),
            pl.BlockSpec((T, 1), lambda: (0, 0)),
            pl.BlockSpec((T, 16), lambda: (0, 0)),
            pl.BlockSpec((T, 16), lambda: (0, 0)),
            pl.BlockSpec((NBLK, 1), lambda: (0, 0)),
            pl.BlockSpec((1, 1), lambda: (0, 0)),
        ],
    )(x, router_w)


def _dispatch_body(x_hbm, pos0_hbm, pos1_hbm, xs_hbm, xbuf, idx0, idx1, sem):
    wid = lax.axis_index("s") * 2 + lax.axis_index("c")
    pltpu.sync_copy(pos0_hbm.at[wid], idx0)
    pltpu.sync_copy(pos1_hbm.at[wid], idx1)
    pltpu.sync_copy(x_hbm.at[pl.ds(wid * TPW, TPW)], xbuf)
    cp0 = pltpu.async_copy(xbuf, xs_hbm.at[idx0], sem)
    cp1 = pltpu.async_copy(xbuf, xs_hbm.at[idx1], sem)
    cp0.wait()
    cp1.wait()


def _dispatch_call():
    return pl.kernel(
        _dispatch_body,
        out_type=jax.ShapeDtypeStruct((NPAD, D), jnp.float32),
        mesh=plsc.VectorSubcoreMesh(core_axis_name="c", subcore_axis_name="s"),
        scratch_types=[
            pltpu.VMEM((TPW, D), jnp.float32),
            pltpu.VMEM((TPW,), jnp.int32),
            pltpu.VMEM((TPW,), jnp.int32),
            pltpu.SemaphoreType.DMA,
        ],
        compiler_params=pltpu.CompilerParams(disable_bounds_checks=True),
    )


def _grouped_body(eid_ref, nact_ref, xs_ref, g_ref, u_ref, d_ref, ys_ref,
                  gbf, ubf, dbf):
    i = pl.program_id(0)

    @pl.when(i < nact_ref[0])
    def _():
        prev = eid_ref[jnp.maximum(i - 1, 0)]
        is_new = jnp.logical_or(i == 0, eid_ref[i] != prev)

        @pl.when(is_new)
        def _():
            gbf[...] = g_ref[0].astype(jnp.bfloat16)
            ubf[...] = u_ref[0].astype(jnp.bfloat16)
            dbf[...] = d_ref[0].astype(jnp.bfloat16)

        xs = xs_ref[...].astype(jnp.bfloat16)
        gate = lax.dot_general(xs, gbf[...], (((1,), (1,)), ((), ())),
                               preferred_element_type=jnp.float32)
        up = lax.dot_general(xs, ubf[...], (((1,), (1,)), ((), ())),
                             preferred_element_type=jnp.float32)
        h = (gate * _sigmoid(gate) * up).astype(jnp.bfloat16)
        y = lax.dot_general(h, dbf[...], (((1,), (1,)), ((), ())),
                            preferred_element_type=jnp.float32)
        ilo = _rne_bf16_bits(y[:, :D // 2])
        ihi = _rne_bf16_bits(y[:, D // 2:])
        ys_ref[...] = ilo | (ihi << 16)


def _grouped_call(eid, nact, xs, gate_w, up_w, down_w):
    grid_spec = pltpu.PrefetchScalarGridSpec(
        num_scalar_prefetch=2,
        grid=(NBLK,),
        in_specs=[
            pl.BlockSpec((BTG, D),
                         lambda i, eid, nact: (jnp.minimum(i, nact[0] - 1), 0)),
            pl.BlockSpec((1, FF, D), lambda i, eid, nact: (eid[i], 0, 0)),
            pl.BlockSpec((1, FF, D), lambda i, eid, nact: (eid[i], 0, 0)),
            pl.BlockSpec((1, D, FF), lambda i, eid, nact: (eid[i], 0, 0)),
        ],
        out_specs=pl.BlockSpec((BTG, D // 2), lambda i, eid, nact: (i, 0)),
        scratch_shapes=[
            pltpu.VMEM((FF, D), jnp.bfloat16),
            pltpu.VMEM((FF, D), jnp.bfloat16),
            pltpu.VMEM((D, FF), jnp.bfloat16),
        ],
    )
    return pl.pallas_call(
        _grouped_body,
        grid_spec=grid_spec,
        out_shape=jax.ShapeDtypeStruct((NPAD, D // 2), jnp.int32),
        compiler_params=pltpu.CompilerParams(
            dimension_semantics=("arbitrary",)),
    )(eid, nact, xs, gate_w, up_w, down_w)


def _shared_body(x_ref, g_ref, u_ref, d_ref, o_ref, gbf, ubf, dbf):
    t = pl.program_id(0)

    @pl.when(t == 0)
    def _():
        gbf[...] = g_ref[...].astype(jnp.bfloat16)
        ubf[...] = u_ref[...].astype(jnp.bfloat16)
        dbf[...] = d_ref[...].astype(jnp.bfloat16)

    x = x_ref[...].astype(jnp.bfloat16)
    gate = lax.dot_general(x, gbf[...], (((1,), (1,)), ((), ())),
                           preferred_element_type=jnp.float32)
    up = lax.dot_general(x, ubf[...], (((1,), (1,)), ((), ())),
                         preferred_element_type=jnp.float32)
    h = (gate * _sigmoid(gate) * up).astype(jnp.bfloat16)
    sh = lax.dot_general(h, dbf[...], (((1,), (1,)), ((), ())),
                         preferred_element_type=jnp.float32)
    ilo = _rne_bf16_bits(sh[:, :D // 2])
    ihi = _rne_bf16_bits(sh[:, D // 2:])
    o_ref[...] = ilo | (ihi << 16)


def _shared_call(x, sh_gate, sh_up, sh_down):
    FFS = sh_gate.shape[0]
    return pl.pallas_call(
        _shared_body,
        grid=(T // BTS,),
        out_shape=jax.ShapeDtypeStruct((T, D // 2), jnp.int32),
        in_specs=[
            pl.BlockSpec((BTS, D), lambda t: (t, 0)),
            pl.BlockSpec((FFS, D), lambda t: (0, 0)),
            pl.BlockSpec((FFS, D), lambda t: (0, 0)),
            pl.BlockSpec((D, FFS), lambda t: (0, 0)),
        ],
        out_specs=pl.BlockSpec((BTS, D // 2), lambda t: (t, 0)),
        scratch_shapes=[
            pltpu.VMEM((FFS, D), jnp.bfloat16),
            pltpu.VMEM((FFS, D), jnp.bfloat16),
            pltpu.VMEM((D, FFS), jnp.bfloat16),
        ],
        compiler_params=pltpu.CompilerParams(
            dimension_semantics=("arbitrary",)),
    )(x, sh_gate, sh_up, sh_down)


def _combine_body(ys_hbm, sh_hbm, pos0_hbm, pos1_hbm, w0_hbm, w1_hbm,
                  out_hbm, y0buf, y1buf, shbuf, obuf, w0buf, w1buf,
                  idx0, idx1, semA, semB):
    wid = lax.axis_index("s") * 2 + lax.axis_index("c")
    nchunks = TPW // CHC
    sems = (semA, semB)

    def issue(c, p):
        sem = sems[p]
        pltpu.sync_copy(pos0_hbm.at[wid, pl.ds(c * CHC, CHC)], idx0.at[p])
        pltpu.sync_copy(pos1_hbm.at[wid, pl.ds(c * CHC, CHC)], idx1.at[p])
        base = wid * TPW + c * CHC
        return [
            pltpu.async_copy(ys_hbm.at[idx0.at[p]], y0buf.at[p], sem),
            pltpu.async_copy(ys_hbm.at[idx1.at[p]], y1buf.at[p], sem),
            pltpu.async_copy(sh_hbm.at[pl.ds(base, CHC)], shbuf.at[p], sem),
            pltpu.async_copy(w0_hbm.at[pl.ds(base, CHC)], w0buf.at[p], sem),
            pltpu.async_copy(w1_hbm.at[pl.ds(base, CHC)], w1buf.at[p], sem),
        ]

    cps = issue(0, 0)
    for c in range(nchunks):
        p = c % 2
        for cp in cps:
            cp.wait()
        nxt = issue(c + 1, 1 - p) if c + 1 < nchunks else None

        @plsc.parallel_loop(0, CHC, 1, unroll=4)
        def _(j, p=p):
            w0v = w0buf[p, j]
            w1v = w1buf[p, j]
            for ci in range(D // 32):
                sl = pl.ds(ci * 16, 16)
                slo = pl.ds(ci * 16, 16)
                shi = pl.ds(D // 2 + ci * 16, 16)
                v0 = y0buf[p, j, sl]
                v1 = y1buf[p, j, sl]
                vs = shbuf[p, j, sl]
                y0lo = plsc.bitcast(lax.shift_left(v0, 16), jnp.float32)
                y1lo = plsc.bitcast(lax.shift_left(v1, 16), jnp.float32)
                slo_f = plsc.bitcast(lax.shift_left(vs, 16), jnp.float32)
                y0hi = plsc.bitcast(v0 & jnp.int32(-65536), jnp.float32)
                y1hi = plsc.bitcast(v1 & jnp.int32(-65536), jnp.float32)
                shi_f = plsc.bitcast(vs & jnp.int32(-65536), jnp.float32)
                obuf[j, slo] = y0lo * w0v + y1lo * w1v + slo_f
                obuf[j, shi] = y0hi * w0v + y1hi * w1v + shi_f
        base = wid * TPW + c * CHC
        pltpu.sync_copy(obuf, out_hbm.at[pl.ds(base, CHC)])
        cps = nxt


def _combine_call():
    return pl.kernel(
        _combine_body,
        out_type=jax.ShapeDtypeStruct((T, D), jnp.float32),
        mesh=plsc.VectorSubcoreMesh(core_axis_name="c", subcore_axis_name="s"),
        scratch_types=[
            pltpu.VMEM((2, CHC, D // 2), jnp.int32),
            pltpu.VMEM((2, CHC, D // 2), jnp.int32),
            pltpu.VMEM((2, CHC, D // 2), jnp.int32),
            pltpu.VMEM((CHC, D), jnp.float32),
            pltpu.VMEM((2, CHC, 16), jnp.float32),
            pltpu.VMEM((2, CHC, 16), jnp.float32),
            pltpu.VMEM((2, CHC), jnp.int32),
            pltpu.VMEM((2, CHC), jnp.int32),
            pltpu.SemaphoreType.DMA,
            pltpu.SemaphoreType.DMA,
        ],
        compiler_params=pltpu.CompilerParams(needs_layout_passes=False,
                                             disable_bounds_checks=True),
    )


def kernel(hidden_states, router_w, gate_w, up_w, down_w, sh_gate, sh_up,
           sh_down):
    x = hidden_states.reshape(T, D)

    pos0, pos1, w0b, w1b, eid, nact = _router_call(x, router_w)
    pos0r = pos0.reshape(NW, TPW)
    pos1r = pos1.reshape(NW, TPW)

    xs = _dispatch_call()(x, pos0r, pos1r)
    ys = _grouped_call(eid.reshape(NBLK), nact.reshape(1), xs,
                       gate_w, up_w, down_w)
    sh = _shared_call(x, sh_gate, sh_up, sh_down)
    out = _combine_call()(ys, sh, pos0r, pos1r, w0b, w1b)

    return out.reshape(hidden_states.shape)
